# baseline (speedup 1.0000x reference)
import functools

import jax
import jax.numpy as jnp
from jax import lax
from jax.experimental import pallas as pl
from jax.experimental.pallas import tpu as pltpu
from jax.experimental.pallas import tpu_sc as plsc

_N = 10000
_E = 5000
_NNZ = 320000
_D = 128

_NC = 2
_NS = 16
_NW = _NC * _NS

_NP = 10240
_EP = 5120
_CHUNK = 80
_IDX_ROWS = _NNZ // _CHUNK
_ROWS_W = _IDX_ROWS // _NW
_INIT_ROWS = 1000
_SEG = 64

_mesh = plsc.VectorSubcoreMesh(core_axis_name="c", subcore_axis_name="s")


@functools.partial(
    pl.kernel,
    out_type=(
        jax.ShapeDtypeStruct((_NP,), jnp.float32),
        jax.ShapeDtypeStruct((_EP,), jnp.float32),
    ),
    mesh=_mesh,
    scratch_types=[
        pltpu.VMEM_SHARED((_NP,), jnp.float32),
        pltpu.VMEM((_ROWS_W, _CHUNK), jnp.int32),
        pltpu.VMEM((_CHUNK,), jnp.float32),
        pltpu.SemaphoreType.DMA,
    ],
)
def _degrees(nidx_hbm, hidx_hbm, zeros_hbm, dv_out, de_out,
             acc_sh, idx_v, ones_v, sem):
    c = lax.axis_index("c")
    s = lax.axis_index("s")

    @pl.when(s == 0)
    def _():
        pltpu.sync_copy(zeros_hbm, acc_sh)

    for i in range(_CHUNK // 16):
        ones_v[pl.ds(16 * i, 16)] = jnp.full((16,), 1.0, jnp.float32)
    plsc.subcore_barrier()

    def fire(j):
        pltpu.async_copy(ones_v, acc_sh.at[idx_v.at[j]], sem, add=True)

    def drain(j):
        pltpu.make_async_copy(ones_v, acc_sh.at[idx_v.at[j]], sem).wait()

    grp = 8
    nfull = _ROWS_W // grp
    tail = _ROWS_W - nfull * grp
    for half in range(2):
        w = s * 2 + half

        @pl.when(c == 0)
        def _():
            pltpu.sync_copy(nidx_hbm.at[w], idx_v)

        @pl.when(c == 1)
        def _():
            pltpu.sync_copy(hidx_hbm.at[w], idx_v)

        for q in range(grp):
            fire(q)

        def body(p, carry):
            for q in range(grp):
                fire(grp * p + q)
            for q in range(grp):
                drain(grp * (p - 1) + q)
            return carry

        lax.fori_loop(1, nfull, body, 0)
        for q in range(grp):
            drain(grp * (nfull - 1) + q)
        for q in range(tail):
            fire(grp * nfull + q)
        for q in range(tail):
            drain(grp * nfull + q)

    plsc.subcore_barrier()

    @pl.when((c == 0) & (s == 0))
    def _():
        pltpu.sync_copy(acc_sh, dv_out)

    @pl.when((c == 1) & (s == 0))
    def _():
        pltpu.sync_copy(acc_sh.at[pl.ds(0, _EP)], de_out)


def _make_agg(acc_rows):
    n_init = acc_rows // _INIT_ROWS

    @functools.partial(
        pl.kernel,
        out_type=jax.ShapeDtypeStruct((_NC, acc_rows, _D), jnp.float32),
        mesh=_mesh,
        scratch_types=[
            pltpu.VMEM_SHARED((acc_rows, _D), jnp.float32),
            pltpu.VMEM((_SEG, _CHUNK), jnp.int32),
            pltpu.VMEM((_SEG, _CHUNK), jnp.int32),
            pltpu.VMEM((_CHUNK, _D), jnp.float32),
            pltpu.VMEM((_CHUNK, _D), jnp.float32),
            pltpu.SemaphoreType.DMA,
            pltpu.SemaphoreType.DMA,
            pltpu.SemaphoreType.DMA,
            pltpu.SemaphoreType.DMA,
        ],
    )
    def _agg(tbl_hbm, gidx_hbm, sidx_hbm, zeros_hbm, out_hbm,
             acc_sh, gidx_v, sidx_v, rows0_v, rows1_v,
             gsem0, gsem1, ssem0, ssem1):
        c = lax.axis_index("c")
        s = lax.axis_index("s")
        wid = s * _NC + c

        @pl.when(s < n_init)
        def _():
            pltpu.sync_copy(
                zeros_hbm.at[pl.ds(s * _INIT_ROWS, _INIT_ROWS)],
                acc_sh.at[pl.ds(s * _INIT_ROWS, _INIT_ROWS)])

        plsc.subcore_barrier()

        bufs = (rows0_v, rows1_v)
        gsems = (gsem0, gsem1)
        ssems = (ssem0, ssem1)

        def start_g(j, k):
            pltpu.async_copy(tbl_hbm.at[gidx_v.at[j]], bufs[k], gsems[k])

        def wait_g(j, k):
            pltpu.make_async_copy(tbl_hbm.at[gidx_v.at[j]], bufs[k],
                                  gsems[k]).wait()

        def scat(j, k):
            pltpu.sync_copy(bufs[k], acc_sh.at[sidx_v.at[j]], add=True)

        def scat_async(j, k):
            return pltpu.async_copy(bufs[k], acc_sh.at[sidx_v.at[j]],
                                    ssems[k], add=True)

        for off, n in ((0, _SEG), (_SEG, _ROWS_W - _SEG)):
            pltpu.sync_copy(gidx_hbm.at[wid, pl.ds(off, n)],
                            gidx_v.at[pl.ds(0, n)])
            pltpu.sync_copy(sidx_hbm.at[wid, pl.ds(off, n)],
                            sidx_v.at[pl.ds(0, n)])
            m = n - (n % 2)
            start_g(0, 0)
            start_g(1, 1)

            def body(p, carry, m=m):
                j = 2 * p
                wait_g(j, 0)
                d0 = scat_async(j, 0)
                wait_g(j + 1, 1)
                d1 = scat_async(j + 1, 1)
                d0.wait()

                @pl.when(j + 2 < m)
                def _():
                    start_g(j + 2, 0)

                d1.wait()

                @pl.when(j + 3 < m)
                def _():
                    start_g(j + 3, 1)

                return carry

            lax.fori_loop(0, m // 2, body, 0)
            if n % 2:
                start_g(n - 1, 0)
                wait_g(n - 1, 0)
                scat(n - 1, 0)

        plsc.subcore_barrier()

        @pl.when(s < n_init)
        def _():
            pltpu.sync_copy(
                acc_sh.at[pl.ds(s * _INIT_ROWS, _INIT_ROWS)],
                out_hbm.at[c, pl.ds(s * _INIT_ROWS, _INIT_ROWS)])

    return _agg


_agg_edges = _make_agg(_E)
_agg_nodes = _make_agg(_N)


def _proj_body(x_ref, w_ref, b_ref, dv_ref, out_ref):
    h = lax.dot_general(x_ref[...], w_ref[...], (((1,), (1,)), ((), ())),
                        preferred_element_type=jnp.float32)
    dv = dv_ref[...]
    scale = jnp.where(dv > 0, lax.rsqrt(dv), 0.0)
    out_ref[...] = (h + b_ref[...]) * scale


def _edge_body(p_ref, de_ref, out_ref):
    de = de_ref[...]
    inv = jnp.where(de > 0, 1.0 / de, 0.0)
    out_ref[...] = (p_ref[0] + p_ref[1]) * inv


def _node_body(p_ref, dv_ref, out_ref):
    dv = dv_ref[...]
    scale = jnp.where(dv > 0, lax.rsqrt(dv), 0.0)
    out_ref[...] = jnp.maximum((p_ref[0] + p_ref[1]) * scale, 0.0)


def kernel(X, node_idx, hedge_idx, W, b):
    nidx2 = node_idx.astype(jnp.int32).reshape(_NW, _ROWS_W, _CHUNK)
    hidx2 = hedge_idx.astype(jnp.int32).reshape(_NW, _ROWS_W, _CHUNK)
    zeros1 = jnp.zeros((_NP,), jnp.float32)
    zeros2 = jnp.zeros((_N, _D), jnp.float32)

    dvp, dep = _degrees(nidx2, hidx2, zeros1)
    dv = dvp[:_N]
    de = dep[:_E]

    hs = pl.pallas_call(
        _proj_body,
        out_shape=jax.ShapeDtypeStruct((_N, _D), jnp.float32),
    )(X, W, b.reshape(1, _D), dv.reshape(_N, 1))

    yep = _agg_edges(hs, nidx2, hidx2, zeros2)

    ye = pl.pallas_call(
        _edge_body,
        out_shape=jax.ShapeDtypeStruct((_E, _D), jnp.float32),
    )(yep, de.reshape(_E, 1))

    zp = _agg_nodes(ye, hidx2, nidx2, zeros2)

    z = pl.pallas_call(
        _node_body,
        out_shape=jax.ShapeDtypeStruct((_N, _D), jnp.float32),
    )(zp, dv.reshape(_N, 1))
    return z


# speedup vs baseline: 1.0560x; 1.0560x over previous
import functools

import jax
import jax.numpy as jnp
from jax import lax
from jax.experimental import pallas as pl
from jax.experimental.pallas import tpu as pltpu
from jax.experimental.pallas import tpu_sc as plsc

_N = 10000
_E = 5000
_NNZ = 320000
_D = 128

_PAD = 8
_NP = 10240
_EP = 5120

_NC = 2
_NS = 16
_NW = _NC * _NS

_CHUNK = 128
_NNZP = 327680
_MAJ = _NNZP // (8 * _CHUNK)
_MAJ_W = _MAJ // _NW
_MAJ_T = _MAJ // _NS
_SEG_MAJ = 2
_NSEG = _MAJ_W // _SEG_MAJ
_SEG_CH = _SEG_MAJ * 8
_INIT_ROWS = 1000

_mesh = plsc.VectorSubcoreMesh(core_axis_name="c", subcore_axis_name="s")


@functools.partial(
    pl.kernel,
    out_type=(
        jax.ShapeDtypeStruct((_NP,), jnp.float32),
        jax.ShapeDtypeStruct((_EP,), jnp.float32),
    ),
    mesh=_mesh,
    scratch_types=[
        pltpu.VMEM_SHARED((_NP,), jnp.float32),
        pltpu.VMEM((_MAJ_T, 8, _CHUNK), jnp.int32),
        pltpu.VMEM((_CHUNK,), jnp.float32),
        pltpu.SemaphoreType.DMA,
    ],
)
def _degrees(nidx_hbm, hidx_hbm, zeros_hbm, dv_out, de_out,
             acc_sh, idx_v, ones_v, sem):
    c = lax.axis_index("c")
    s = lax.axis_index("s")

    @pl.when(s == 0)
    def _():
        pltpu.sync_copy(zeros_hbm, acc_sh)

    for i in range(_CHUNK // 16):
        ones_v[pl.ds(16 * i, 16)] = jnp.full((16,), 1.0, jnp.float32)

    @pl.when(c == 0)
    def _():
        pltpu.sync_copy(nidx_hbm.at[pl.ds(s * _MAJ_T, _MAJ_T)], idx_v)

    @pl.when(c == 1)
    def _():
        pltpu.sync_copy(hidx_hbm.at[pl.ds(s * _MAJ_T, _MAJ_T)], idx_v)

    plsc.subcore_barrier()

    def fire(a, b):
        pltpu.async_copy(ones_v, acc_sh.at[idx_v.at[a, b]], sem, add=True)

    def drain(a, b):
        pltpu.make_async_copy(ones_v, acc_sh.at[idx_v.at[a, b]], sem).wait()

    for b in range(8):
        fire(0, b)

    def body(a, carry):
        for b in range(8):
            fire(a, b)
        for b in range(8):
            drain(a - 1, b)
        return carry

    lax.fori_loop(1, _MAJ_T, body, 0)
    for b in range(8):
        drain(_MAJ_T - 1, b)

    plsc.subcore_barrier()

    @pl.when((c == 0) & (s == 0))
    def _():
        pltpu.sync_copy(acc_sh, dv_out)

    @pl.when((c == 1) & (s == 0))
    def _():
        pltpu.sync_copy(acc_sh.at[pl.ds(0, _EP)], de_out)


def _make_agg(acc_rows):
    n_init = acc_rows // _INIT_ROWS

    @functools.partial(
        pl.kernel,
        out_type=jax.ShapeDtypeStruct((_NC, acc_rows, _D), jnp.float32),
        mesh=_mesh,
        scratch_types=[
            pltpu.VMEM_SHARED((acc_rows + _PAD, _D), jnp.float32),
            pltpu.VMEM((_SEG_MAJ, 8, _CHUNK), jnp.int32),
            pltpu.VMEM((_SEG_MAJ, 8, _CHUNK), jnp.int32),
            pltpu.VMEM((_CHUNK, _D), jnp.float32),
            pltpu.VMEM((_CHUNK, _D), jnp.float32),
            pltpu.SemaphoreType.DMA,
            pltpu.SemaphoreType.DMA,
        ],
    )
    def _agg(tbl_hbm, gidx_hbm, sidx_hbm, zeros_hbm, out_hbm,
             acc_sh, gidx_v, sidx_v, rows0_v, rows1_v, gsem0, gsem1):
        c = lax.axis_index("c")
        s = lax.axis_index("s")
        wid = s * _NC + c

        @pl.when(s < n_init)
        def _():
            pltpu.sync_copy(
                zeros_hbm.at[pl.ds(s * _INIT_ROWS, _INIT_ROWS)],
                acc_sh.at[pl.ds(s * _INIT_ROWS, _INIT_ROWS)])

        plsc.subcore_barrier()

        bufs = (rows0_v, rows1_v)
        gsems = (gsem0, gsem1)

        def start_g(q, k):
            pltpu.async_copy(tbl_hbm.at[gidx_v.at[q // 8, q % 8]],
                             bufs[k], gsems[k])

        def wait_g(q, k):
            pltpu.make_async_copy(tbl_hbm.at[gidx_v.at[q // 8, q % 8]],
                                  bufs[k], gsems[k]).wait()

        def scat(q, k):
            pltpu.sync_copy(bufs[k], acc_sh.at[sidx_v.at[q // 8, q % 8]],
                            add=True)

        def seg_body(seg, carry):
            base = wid * _MAJ_W + seg * _SEG_MAJ
            pltpu.sync_copy(gidx_hbm.at[pl.ds(base, _SEG_MAJ)], gidx_v)
            pltpu.sync_copy(sidx_hbm.at[pl.ds(base, _SEG_MAJ)], sidx_v)
            start_g(0, 0)

            def body(p, carry2):
                q = 2 * p
                start_g(q + 1, 1)
                wait_g(q, 0)
                scat(q, 0)

                @pl.when(q + 2 < _SEG_CH)
                def _():
                    start_g(q + 2, 0)

                wait_g(q + 1, 1)
                scat(q + 1, 1)
                return carry2

            lax.fori_loop(0, _SEG_CH // 2, body, 0)
            return carry

        lax.fori_loop(0, _NSEG, seg_body, 0)
        plsc.subcore_barrier()

        @pl.when(s < n_init)
        def _():
            pltpu.sync_copy(
                acc_sh.at[pl.ds(s * _INIT_ROWS, _INIT_ROWS)],
                out_hbm.at[c, pl.ds(s * _INIT_ROWS, _INIT_ROWS)])

    return _agg


_agg_edges = _make_agg(_E)
_agg_nodes = _make_agg(_N)


def _proj_body(x_ref, w_ref, b_ref, dv_ref, out_ref):
    h = lax.dot_general(x_ref[...], w_ref[...], (((1,), (1,)), ((), ())),
                        preferred_element_type=jnp.float32)
    dv = dv_ref[...]
    scale = jnp.where(dv > 0, lax.rsqrt(dv), 0.0)
    hs = (h + b_ref[...]) * scale
    out_ref[...] = jnp.concatenate(
        [hs, jnp.zeros((_PAD, _D), jnp.float32)], axis=0)


def _edge_body(p_ref, de_ref, out_ref):
    de = de_ref[...]
    inv = jnp.where(de > 0, 1.0 / de, 0.0)
    ye = (p_ref[0] + p_ref[1]) * inv
    out_ref[...] = jnp.concatenate(
        [ye, jnp.zeros((_PAD, _D), jnp.float32)], axis=0)


def _node_body(p_ref, dv_ref, out_ref):
    dv = dv_ref[...]
    scale = jnp.where(dv > 0, lax.rsqrt(dv), 0.0)
    out_ref[...] = jnp.maximum((p_ref[0] + p_ref[1]) * scale, 0.0)


def kernel(X, node_idx, hedge_idx, W, b):
    npad = _NNZP - _NNZ
    lane = jnp.arange(npad, dtype=jnp.int32) % _PAD
    nidx3 = jnp.concatenate(
        [node_idx.astype(jnp.int32), _N + lane]).reshape(_MAJ, 8, _CHUNK)
    hidx3 = jnp.concatenate(
        [hedge_idx.astype(jnp.int32), _E + lane]).reshape(_MAJ, 8, _CHUNK)
    zeros1 = jnp.zeros((_NP,), jnp.float32)
    zeros2 = jnp.zeros((_N, _D), jnp.float32)

    dvp, dep = _degrees(nidx3, hidx3, zeros1)
    dv = dvp[:_N]
    de = dep[:_E]

    hs = pl.pallas_call(
        _proj_body,
        out_shape=jax.ShapeDtypeStruct((_N + _PAD, _D), jnp.float32),
    )(X, W, b.reshape(1, _D), dv.reshape(_N, 1))

    yep = _agg_edges(hs, nidx3, hidx3, zeros2)

    ye = pl.pallas_call(
        _edge_body,
        out_shape=jax.ShapeDtypeStruct((_E + _PAD, _D), jnp.float32),
    )(yep, de.reshape(_E, 1))

    zp = _agg_nodes(ye, hidx3, nidx3, zeros2)

    z = pl.pallas_call(
        _node_body,
        out_shape=jax.ShapeDtypeStruct((_N, _D), jnp.float32),
    )(zp, dv.reshape(_N, 1))
    return z


# speedup vs baseline: 1.2629x; 1.1959x over previous
import functools

import jax
import jax.numpy as jnp
from jax import lax
from jax.experimental import pallas as pl
from jax.experimental.pallas import tpu as pltpu
from jax.experimental.pallas import tpu_sc as plsc

_N = 10000
_E = 5000
_NNZ = 320000
_D = 128

_PAD = 8
_NP = 10240
_EP = 5120

_NC = 2
_NS = 16
_NW = _NC * _NS

_CHUNK = 128
_NNZP = 327680
_MAJ = _NNZP // (8 * _CHUNK)
_MAJ_W = _MAJ // _NW
_MAJ_T = _MAJ // _NS
_SEG_MAJ = 2
_NSEG = _MAJ_W // _SEG_MAJ
_SEG_CH = _SEG_MAJ * 8
_NCH_REAL = _NNZ // _CHUNK
_INIT_ROWS = 1000

_mesh = plsc.VectorSubcoreMesh(core_axis_name="c", subcore_axis_name="s")


@functools.partial(
    pl.kernel,
    out_type=(
        jax.ShapeDtypeStruct((_NP,), jnp.float32),
        jax.ShapeDtypeStruct((_EP,), jnp.float32),
    ),
    mesh=_mesh,
    scratch_types=[
        pltpu.VMEM_SHARED((_NP,), jnp.float32),
        pltpu.VMEM((_MAJ_T, 8, _CHUNK), jnp.int32),
        pltpu.VMEM((_CHUNK,), jnp.float32),
        pltpu.SemaphoreType.DMA,
    ],
)
def _degrees(nidx_hbm, hidx_hbm, zeros_hbm, dv_out, de_out,
             acc_sh, idx_v, ones_v, sem):
    c = lax.axis_index("c")
    s = lax.axis_index("s")

    @pl.when(s == 0)
    def _():
        pltpu.sync_copy(zeros_hbm, acc_sh)

    for i in range(_CHUNK // 16):
        ones_v[pl.ds(16 * i, 16)] = jnp.full((16,), 1.0, jnp.float32)

    @pl.when(c == 0)
    def _():
        pltpu.sync_copy(nidx_hbm.at[pl.ds(s * _MAJ_T, _MAJ_T)], idx_v)

    @pl.when(c == 1)
    def _():
        pltpu.sync_copy(hidx_hbm.at[pl.ds(s * _MAJ_T, _MAJ_T)], idx_v)

    plsc.subcore_barrier()

    def real(a, b):
        return (s * _MAJ_T + a) * 8 + b < _NCH_REAL

    def fire(a, b):
        @pl.when(real(a, b))
        def _():
            pltpu.async_copy(ones_v, acc_sh.at[idx_v.at[a, b]], sem,
                             add=True)

    def drain(a, b):
        @pl.when(real(a, b))
        def _():
            pltpu.make_async_copy(ones_v, acc_sh.at[idx_v.at[a, b]],
                                  sem).wait()

    for b in range(8):
        fire(0, b)

    def body(a, carry):
        for b in range(8):
            fire(a, b)
        for b in range(8):
            drain(a - 1, b)
        return carry

    lax.fori_loop(1, _MAJ_T, body, 0)
    for b in range(8):
        drain(_MAJ_T - 1, b)

    plsc.subcore_barrier()

    @pl.when((c == 0) & (s == 0))
    def _():
        pltpu.sync_copy(acc_sh, dv_out)

    @pl.when((c == 1) & (s == 0))
    def _():
        pltpu.sync_copy(acc_sh.at[pl.ds(0, _EP)], de_out)


def _make_agg(acc_rows):
    n_init = acc_rows // _INIT_ROWS

    @functools.partial(
        pl.kernel,
        out_type=jax.ShapeDtypeStruct((_NC, acc_rows, _D), jnp.float32),
        mesh=_mesh,
        scratch_types=[
            pltpu.VMEM_SHARED((acc_rows, _D), jnp.float32),
            pltpu.VMEM((_SEG_MAJ, 8, _CHUNK), jnp.int32),
            pltpu.VMEM((_SEG_MAJ, 8, _CHUNK), jnp.int32),
            pltpu.VMEM((_CHUNK, _D), jnp.float32),
            pltpu.VMEM((_CHUNK, _D), jnp.float32),
            pltpu.SemaphoreType.DMA,
            pltpu.SemaphoreType.DMA,
        ],
    )
    def _agg(tbl_hbm, gidx_hbm, sidx_hbm, zeros_hbm, out_hbm,
             acc_sh, gidx_v, sidx_v, rows0_v, rows1_v, gsem0, gsem1):
        c = lax.axis_index("c")
        s = lax.axis_index("s")
        wid = s * _NC + c

        @pl.when(s < n_init)
        def _():
            pltpu.sync_copy(
                zeros_hbm.at[pl.ds(s * _INIT_ROWS, _INIT_ROWS)],
                acc_sh.at[pl.ds(s * _INIT_ROWS, _INIT_ROWS)])

        plsc.subcore_barrier()

        bufs = (rows0_v, rows1_v)
        gsems = (gsem0, gsem1)

        def start_g(q, k):
            pltpu.async_copy(tbl_hbm.at[gidx_v.at[q // 8, q % 8]],
                             bufs[k], gsems[k])

        def wait_g(q, k):
            pltpu.make_async_copy(tbl_hbm.at[gidx_v.at[q // 8, q % 8]],
                                  bufs[k], gsems[k]).wait()

        def scat(q, k, base_ch):
            @pl.when(base_ch + q < _NCH_REAL)
            def _():
                pltpu.sync_copy(bufs[k], acc_sh.at[sidx_v.at[q // 8, q % 8]],
                                add=True)

        def seg_body(seg, carry):
            base = wid * _MAJ_W + seg * _SEG_MAJ
            base_ch = base * 8
            pltpu.sync_copy(gidx_hbm.at[pl.ds(base, _SEG_MAJ)], gidx_v)
            pltpu.sync_copy(sidx_hbm.at[pl.ds(base, _SEG_MAJ)], sidx_v)
            start_g(0, 0)

            def body(p, carry2):
                q = 2 * p
                start_g(q + 1, 1)
                wait_g(q, 0)
                scat(q, 0, base_ch)

                @pl.when(q + 2 < _SEG_CH)
                def _():
                    start_g(q + 2, 0)

                wait_g(q + 1, 1)
                scat(q + 1, 1, base_ch)
                return carry2

            lax.fori_loop(0, _SEG_CH // 2, body, 0)
            return carry

        lax.fori_loop(0, _NSEG, seg_body, 0)
        plsc.subcore_barrier()

        @pl.when(s < n_init)
        def _():
            pltpu.sync_copy(
                acc_sh.at[pl.ds(s * _INIT_ROWS, _INIT_ROWS)],
                out_hbm.at[c, pl.ds(s * _INIT_ROWS, _INIT_ROWS)])

    return _agg


_agg_edges = _make_agg(_E)
_agg_nodes = _make_agg(_N)


def _proj_body(x_ref, w_ref, b_ref, dv_ref, out_ref):
    h = lax.dot_general(x_ref[...], w_ref[...], (((1,), (1,)), ((), ())),
                        preferred_element_type=jnp.float32)
    dv = dv_ref[...]
    scale = jnp.where(dv > 0, lax.rsqrt(dv), 0.0)
    out_ref[...] = (h + b_ref[...]) * scale


def _edge_body(p_ref, de_ref, out_ref):
    de = de_ref[...]
    inv = jnp.where(de > 0, 1.0 / de, 0.0)
    out_ref[...] = (p_ref[0] + p_ref[1]) * inv


def _node_body(p_ref, dv_ref, out_ref):
    dv = dv_ref[...]
    scale = jnp.where(dv > 0, lax.rsqrt(dv), 0.0)
    out_ref[...] = jnp.maximum((p_ref[0] + p_ref[1]) * scale, 0.0)


def kernel(X, node_idx, hedge_idx, W, b):
    npad = _NNZP - _NNZ
    span = jnp.arange(npad, dtype=jnp.int32)
    nidx3 = jnp.concatenate(
        [node_idx.astype(jnp.int32), span % _N]).reshape(_MAJ, 8, _CHUNK)
    hidx3 = jnp.concatenate(
        [hedge_idx.astype(jnp.int32), span % _E]).reshape(_MAJ, 8, _CHUNK)
    zeros1 = jnp.zeros((_NP,), jnp.float32)
    zeros2 = jnp.zeros((_N, _D), jnp.float32)

    dvp, dep = _degrees(nidx3, hidx3, zeros1)
    dv = dvp[:_N]
    de = dep[:_E]

    hs = pl.pallas_call(
        _proj_body,
        out_shape=jax.ShapeDtypeStruct((_N, _D), jnp.float32),
    )(X, W, b.reshape(1, _D), dv.reshape(_N, 1))

    yep = _agg_edges(hs, nidx3, hidx3, zeros2)

    ye = pl.pallas_call(
        _edge_body,
        out_shape=jax.ShapeDtypeStruct((_E, _D), jnp.float32),
    )(yep, de.reshape(_E, 1))

    zp = _agg_nodes(ye, hidx3, nidx3, zeros2)

    z = pl.pallas_call(
        _node_body,
        out_shape=jax.ShapeDtypeStruct((_N, _D), jnp.float32),
    )(zp, dv.reshape(_N, 1))
    return z


# speedup vs baseline: 1.2656x; 1.0022x over previous
import functools

import jax
import jax.numpy as jnp
from jax import lax
from jax.experimental import pallas as pl
from jax.experimental.pallas import tpu as pltpu
from jax.experimental.pallas import tpu_sc as plsc

_N = 10000
_E = 5000
_NNZ = 320000
_D = 128

_NP = 10240
_EP = 5120

_NC = 2
_NS = 16
_NW = _NC * _NS

_CHUNK = 128
_NNZP = 327680
_MAJ = _NNZP // (8 * _CHUNK)
_MAJ_W = _MAJ // _NW
_MAJ_T = _MAJ // _NS
_SEG_MAJ = 2
_NSEG = _MAJ_W // _SEG_MAJ
_SEG_CH = _SEG_MAJ * 8
_NCH_REAL = _NNZ // _CHUNK
_INIT_ROWS = 1000

_mesh = plsc.VectorSubcoreMesh(core_axis_name="c", subcore_axis_name="s")


@functools.partial(
    pl.kernel,
    out_type=(
        jax.ShapeDtypeStruct((_NP,), jnp.float32),
        jax.ShapeDtypeStruct((_EP,), jnp.float32),
    ),
    mesh=_mesh,
    scratch_types=[
        pltpu.VMEM_SHARED((_NP,), jnp.float32),
        pltpu.VMEM((_MAJ_T, 8, _CHUNK), jnp.int32),
        pltpu.VMEM((_CHUNK,), jnp.float32),
        pltpu.SemaphoreType.DMA,
    ],
)
def _degrees(nidx_hbm, hidx_hbm, zeros_hbm, dv_out, de_out,
             acc_sh, idx_v, ones_v, sem):
    c = lax.axis_index("c")
    s = lax.axis_index("s")

    @pl.when(s == 0)
    def _():
        pltpu.sync_copy(zeros_hbm, acc_sh)

    for i in range(_CHUNK // 16):
        ones_v[pl.ds(16 * i, 16)] = jnp.full((16,), 1.0, jnp.float32)

    @pl.when(c == 0)
    def _():
        pltpu.sync_copy(nidx_hbm.at[pl.ds(s * _MAJ_T, _MAJ_T)], idx_v)

    @pl.when(c == 1)
    def _():
        pltpu.sync_copy(hidx_hbm.at[pl.ds(s * _MAJ_T, _MAJ_T)], idx_v)

    plsc.subcore_barrier()

    def real(a, b):
        return (s * _MAJ_T + a) * 8 + b < _NCH_REAL

    def fire(a, b):
        @pl.when(real(a, b))
        def _():
            pltpu.async_copy(ones_v, acc_sh.at[idx_v.at[a, b]], sem,
                             add=True)

    def drain(a, b):
        @pl.when(real(a, b))
        def _():
            pltpu.make_async_copy(ones_v, acc_sh.at[idx_v.at[a, b]],
                                  sem).wait()

    for b in range(8):
        fire(0, b)

    def body(a, carry):
        for b in range(8):
            fire(a, b)
        for b in range(8):
            drain(a - 1, b)
        return carry

    lax.fori_loop(1, _MAJ_T, body, 0)
    for b in range(8):
        drain(_MAJ_T - 1, b)

    plsc.subcore_barrier()

    @pl.when((c == 0) & (s == 0))
    def _():
        pltpu.sync_copy(acc_sh, dv_out)

    @pl.when((c == 1) & (s == 0))
    def _():
        pltpu.sync_copy(acc_sh.at[pl.ds(0, _EP)], de_out)


def _make_agg(acc_rows):
    n_init = acc_rows // _INIT_ROWS

    @functools.partial(
        pl.kernel,
        out_type=jax.ShapeDtypeStruct((_NC, acc_rows, _D), jnp.float32),
        mesh=_mesh,
        scratch_types=[
            pltpu.VMEM_SHARED((acc_rows, _D), jnp.float32),
            pltpu.VMEM((_SEG_MAJ, 8, _CHUNK), jnp.int32),
            pltpu.VMEM((_SEG_MAJ, 8, _CHUNK), jnp.int32),
            pltpu.VMEM((_CHUNK, _D), jnp.float32),
            pltpu.VMEM((_CHUNK, _D), jnp.float32),
            pltpu.SemaphoreType.DMA,
            pltpu.SemaphoreType.DMA,
        ],
    )
    def _agg(tbl_hbm, gidx_hbm, sidx_hbm, zeros_hbm, out_hbm,
             acc_sh, gidx_v, sidx_v, rows0_v, rows1_v, gsem0, gsem1):
        c = lax.axis_index("c")
        s = lax.axis_index("s")
        wid = s * _NC + c

        @pl.when(s < n_init)
        def _():
            pltpu.sync_copy(
                zeros_hbm.at[pl.ds(s * _INIT_ROWS, _INIT_ROWS)],
                acc_sh.at[pl.ds(s * _INIT_ROWS, _INIT_ROWS)])

        plsc.subcore_barrier()

        bufs = (rows0_v, rows1_v)
        gsems = (gsem0, gsem1)

        def start_g(q, k):
            pltpu.async_copy(tbl_hbm.at[gidx_v.at[q // 8, q % 8]],
                             bufs[k], gsems[k])

        def wait_g(q, k):
            pltpu.make_async_copy(tbl_hbm.at[gidx_v.at[q // 8, q % 8]],
                                  bufs[k], gsems[k]).wait()

        def scat(q, k, base_ch):
            @pl.when(base_ch + q < _NCH_REAL)
            def _():
                pltpu.sync_copy(bufs[k], acc_sh.at[sidx_v.at[q // 8, q % 8]],
                                add=True)

        def seg_body(seg, carry):
            base = wid * _MAJ_W + seg * _SEG_MAJ
            base_ch = base * 8
            pltpu.sync_copy(gidx_hbm.at[pl.ds(base, _SEG_MAJ)], gidx_v)
            pltpu.sync_copy(sidx_hbm.at[pl.ds(base, _SEG_MAJ)], sidx_v)
            start_g(0, 0)

            def body(p, carry2):
                q = 2 * p
                start_g(q + 1, 1)
                wait_g(q, 0)
                scat(q, 0, base_ch)

                @pl.when(q + 2 < _SEG_CH)
                def _():
                    start_g(q + 2, 0)

                wait_g(q + 1, 1)
                scat(q + 1, 1, base_ch)
                return carry2

            lax.fori_loop(0, _SEG_CH // 2, body, 0)
            return carry

        lax.fori_loop(0, _NSEG, seg_body, 0)
        plsc.subcore_barrier()

        @pl.when(s < n_init)
        def _():
            pltpu.sync_copy(
                acc_sh.at[pl.ds(s * _INIT_ROWS, _INIT_ROWS)],
                out_hbm.at[c, pl.ds(s * _INIT_ROWS, _INIT_ROWS)])

    return _agg


_agg_edges = _make_agg(_E)
_agg_nodes = _make_agg(_N)


def _proj_body(x_ref, w_ref, b_ref, dv_ref, out_ref):
    h = lax.dot_general(x_ref[...], w_ref[...], (((1,), (1,)), ((), ())),
                        preferred_element_type=jnp.float32)
    dv = dv_ref[...]
    scale = jnp.where(dv > 0, lax.rsqrt(dv), 0.0)
    out_ref[...] = (h + b_ref[...]) * scale


def _edge_body(p_ref, de_ref, out_ref):
    de = de_ref[...]
    inv = jnp.where(de > 0, 1.0 / de, 0.0)
    out_ref[...] = (p_ref[0] + p_ref[1]) * inv


def _node_body(p_ref, dv_ref, out_ref):
    dv = dv_ref[...]
    scale = jnp.where(dv > 0, lax.rsqrt(dv), 0.0)
    out_ref[...] = jnp.maximum((p_ref[0] + p_ref[1]) * scale, 0.0)


def kernel(X, node_idx, hedge_idx, W, b):
    npad = _NNZP - _NNZ
    span = jnp.arange(npad, dtype=jnp.int32)
    nidx3 = jnp.concatenate(
        [node_idx.astype(jnp.int32), span % _N]).reshape(_MAJ, 8, _CHUNK)
    hidx3 = jnp.concatenate(
        [hedge_idx.astype(jnp.int32), span % _E]).reshape(_MAJ, 8, _CHUNK)
    zeros1 = jnp.zeros((_NP,), jnp.float32)
    zeros2 = jnp.zeros((_N, _D), jnp.float32)

    dvp, dep = _degrees(nidx3, hidx3, zeros1)
    dv = dvp[:_N]
    de = dep[:_E]

    hs = pl.pallas_call(
        _proj_body,
        out_shape=jax.ShapeDtypeStruct((_N, _D), jnp.float32),
    )(X, W, b.reshape(1, _D), dv.reshape(_N, 1))

    yep = _agg_edges(hs, nidx3, hidx3, zeros2)

    ye = pl.pallas_call(
        _edge_body,
        out_shape=jax.ShapeDtypeStruct((_E, _D), jnp.float32),
    )(yep, de.reshape(_E, 1))

    zp = _agg_nodes(ye, hidx3, nidx3, zeros2)

    z = pl.pallas_call(
        _node_body,
        out_shape=jax.ShapeDtypeStruct((_N, _D), jnp.float32),
    )(zp, dv.reshape(_N, 1))
    return z
